# baseline (speedup 1.0000x reference)
import functools

import jax
import jax.numpy as jnp
from jax import lax
from jax.experimental import pallas as pl
from jax.experimental.pallas import tpu as pltpu
from jax.experimental.pallas import tpu_sc as plsc

_B = 16384
_D = 32
_CH = 256
_VE4 = 250000

_info = plsc.get_sparse_core_info()
_NC = _info.num_cores
_NS = _info.num_subcores
_NW = _NC * _NS
_BPW = _B // _NW

_mesh = plsc.VectorSubcoreMesh(core_axis_name="c", subcore_axis_name="s")


@functools.partial(
    pl.kernel,
    mesh=_mesh,
    out_type=jax.ShapeDtypeStruct((4 * _B, _D), jnp.float32),
    compiler_params=pltpu.CompilerParams(use_tc_tiling_on_sc=False),
    scratch_types=[
        pltpu.VMEM((_BPW,), jnp.int32),
        pltpu.VMEM((_BPW,), jnp.int32),
        pltpu.VMEM((_BPW,), jnp.int32),
        pltpu.VMEM((_BPW,), jnp.int32),
        pltpu.VMEM((_CH,), jnp.int32),
        pltpu.VMEM((_BPW, _D), jnp.float32),
        pltpu.VMEM((_BPW, _D), jnp.float32),
        pltpu.VMEM((_BPW, _D), jnp.float32),
        pltpu.VMEM((_CH, 4 * _D), jnp.float32),
        pltpu.VMEM((_BPW, _D), jnp.float32),
        pltpu.SemaphoreType.DMA,
        pltpu.SemaphoreType.DMA,
        pltpu.SemaphoreType.DMA,
        pltpu.SemaphoreType.DMA,
        pltpu.SemaphoreType.DMA,
    ],
)
def _emb_kernel(cat_i, sub_i, elem_i, evt_i,
                ct, st, te4, vt, out,
                i0, i1, i2, i3, ip, r0, r1, r3, wide, row2,
                g0, g1, g3, ge, ws):
    wid = lax.axis_index("s") * _NC + lax.axis_index("c")
    base = wid * _BPW
    pltpu.sync_copy(cat_i.at[pl.ds(base, _BPW)], i0)
    pltpu.sync_copy(sub_i.at[pl.ds(base, _BPW)], i1)
    pltpu.sync_copy(elem_i.at[pl.ds(base, _BPW)], i2)
    pltpu.sync_copy(evt_i.at[pl.ds(base, _BPW)], i3)
    c0 = pltpu.async_copy(ct.at[i0], r0, g0)
    c1 = pltpu.async_copy(st.at[i1], r1, g1)
    c3 = pltpu.async_copy(vt.at[i3], r3, g3)
    for k in range(_BPW // _CH):
        def mkp(g, carry, k=k):
            rv = i2[pl.ds(k * _CH + g * 16, 16)]
            ip[pl.ds(g * 16, 16)] = lax.shift_right_logical(rv, 2)
            return carry
        lax.fori_loop(0, _CH // 16, mkp, 0)
        pltpu.async_copy(te4.at[ip], wide, ge).wait()
        def ext(g, carry, k=k):
            rv = i2[pl.ds(k * _CH + g * 16, 16)]
            for l in range(16):
                q = rv[l] & 3
                src = g * 16 + l
                dst = k * _CH + g * 16 + l
                row2[dst, pl.ds(0, 16)] = wide[src, pl.ds(q * _D, 16)]
                row2[dst, pl.ds(16, 16)] = wide[src, pl.ds(q * _D + 16, 16)]
            return carry
        lax.fori_loop(0, _CH // 16, ext, 0)
    w2 = pltpu.async_copy(row2, out.at[pl.ds(2 * _B + base, _BPW)], ws)
    c0.wait()
    w0 = pltpu.async_copy(r0, out.at[pl.ds(0 * _B + base, _BPW)], ws)
    c1.wait()
    w1 = pltpu.async_copy(r1, out.at[pl.ds(1 * _B + base, _BPW)], ws)
    c3.wait()
    w3 = pltpu.async_copy(r3, out.at[pl.ds(3 * _B + base, _BPW)], ws)
    w0.wait()
    w1.wait()
    w2.wait()
    w3.wait()


def kernel(categories, sub_categories, elements, event_types,
           category_table, sub_category_table, element_table,
           event_type_table):
    cat_i = jnp.asarray(categories, jnp.int32)
    sub_i = jnp.asarray(sub_categories, jnp.int32)
    elem_i = jnp.asarray(elements, jnp.int32)
    evt_i = jnp.asarray(event_types, jnp.int32)
    te4 = element_table.reshape(_VE4, 4 * _D)
    return _emb_kernel(cat_i, sub_i, elem_i, evt_i,
                       category_table, sub_category_table,
                       te4, event_type_table)


# speedup vs baseline: 1.0109x; 1.0109x over previous
import functools

import jax
import jax.numpy as jnp
from jax import lax
from jax.experimental import pallas as pl
from jax.experimental.pallas import tpu as pltpu
from jax.experimental.pallas import tpu_sc as plsc

_B = 16384
_D = 32

_info = plsc.get_sparse_core_info()
_NC = _info.num_cores
_NS = _info.num_subcores
_NW = _NC * _NS
_BPW = _B // _NW

_mesh = plsc.VectorSubcoreMesh(core_axis_name="c", subcore_axis_name="s")


@functools.partial(
    pl.kernel,
    mesh=_mesh,
    out_type=jax.ShapeDtypeStruct((4 * _B, _D), jnp.float32),
    compiler_params=pltpu.CompilerParams(use_tc_tiling_on_sc=False,
                                         needs_layout_passes=False),
    scratch_types=[
        pltpu.VMEM((_BPW,), jnp.int32),
        pltpu.VMEM((_BPW,), jnp.int32),
        pltpu.VMEM((_BPW,), jnp.int32),
        pltpu.VMEM((_BPW,), jnp.int32),
        pltpu.VMEM((_BPW, _D), jnp.float32),
        pltpu.VMEM((_BPW, _D), jnp.float32),
        pltpu.VMEM((_BPW, _D), jnp.float32),
        pltpu.VMEM((_BPW, _D), jnp.float32),
        pltpu.SemaphoreType.DMA,
        pltpu.SemaphoreType.DMA,
        pltpu.SemaphoreType.DMA,
        pltpu.SemaphoreType.DMA,
        pltpu.SemaphoreType.DMA,
    ],
)
def _emb_kernel(cat_i, sub_i, elem_i, evt_i,
                ct, st, et, vt, out,
                i0, i1, i2, i3, r0, r1, r3, r2,
                g0, g1, g3, ge, ws):
    wid = lax.axis_index("s") * _NC + lax.axis_index("c")
    base = wid * _BPW
    pltpu.sync_copy(cat_i.at[pl.ds(base, _BPW)], i0)
    pltpu.sync_copy(sub_i.at[pl.ds(base, _BPW)], i1)
    pltpu.sync_copy(elem_i.at[pl.ds(base, _BPW)], i2)
    pltpu.sync_copy(evt_i.at[pl.ds(base, _BPW)], i3)
    c0 = pltpu.async_copy(ct.at[i0], r0, g0)
    c1 = pltpu.async_copy(st.at[i1], r1, g1)
    c3 = pltpu.async_copy(vt.at[i3], r3, g3)
    c2 = pltpu.async_copy(et.at[i2], r2, ge)
    c0.wait()
    w0 = pltpu.async_copy(r0, out.at[pl.ds(0 * _B + base, _BPW)], ws)
    c1.wait()
    w1 = pltpu.async_copy(r1, out.at[pl.ds(1 * _B + base, _BPW)], ws)
    c2.wait()
    w2 = pltpu.async_copy(r2, out.at[pl.ds(2 * _B + base, _BPW)], ws)
    c3.wait()
    w3 = pltpu.async_copy(r3, out.at[pl.ds(3 * _B + base, _BPW)], ws)
    w0.wait()
    w1.wait()
    w3.wait()
    w2.wait()


def kernel(categories, sub_categories, elements, event_types,
           category_table, sub_category_table, element_table,
           event_type_table):
    cat_i = jnp.asarray(categories, jnp.int32)
    sub_i = jnp.asarray(sub_categories, jnp.int32)
    elem_i = jnp.asarray(elements, jnp.int32)
    evt_i = jnp.asarray(event_types, jnp.int32)
    return _emb_kernel(cat_i, sub_i, elem_i, evt_i,
                       category_table, sub_category_table,
                       element_table, event_type_table)
